# baseline (speedup 1.0000x reference)
import jax
import jax.numpy as jnp
from jax.experimental import pallas as pl

N = 10000
R = 8
H = 128
L = 16


def _final_body(acc_ref, dense_ref, out_ref):
    out_ref[...] = jax.nn.sigmoid(acc_ref[...] + dense_ref[...])


def _mean_agg(msgs, dst, edge_type, num_nodes, num_rel, out_dim):
    keys = dst * num_rel + edge_type
    sums = jax.ops.segment_sum(msgs, keys, num_segments=num_nodes * num_rel)
    counts = jax.ops.segment_sum(
        jnp.ones((msgs.shape[0],), dtype=msgs.dtype), keys,
        num_segments=num_nodes * num_rel)
    mean = sums / jnp.maximum(counts, 1.0)[:, None]
    return mean.reshape(num_nodes, num_rel, out_dim).sum(axis=1)


def kernel(edge_index, edge_type, weight1, root1, bias1, weight2, root2, bias2):
    src = edge_index[0]
    dst = edge_index[1]
    msg1 = weight1[edge_type, src]
    h = _mean_agg(msg1, dst, edge_type, N, R, H)
    h = jax.nn.relu(h + root1 + bias1)
    xw = jnp.einsum('nh,rho->nro', h, weight2)
    msg2 = xw[src, edge_type]
    acc = _mean_agg(msg2, dst, edge_type, N, R, L)
    dense = h @ root2 + bias2
    out = pl.pallas_call(
        _final_body,
        out_shape=jax.ShapeDtypeStruct((N, L), jnp.float32),
    )(acc, dense)
    return out


# speedup vs baseline: 13.6070x; 13.6070x over previous
import jax
import jax.numpy as jnp
from jax import lax
from jax.experimental import pallas as pl
from jax.experimental.pallas import tpu as pltpu
from jax.experimental.pallas import tpu_sc as plsc

N = 10000
R = 8
H = 128
L = 16
E = 320000

TROW = R * N
KS = 80128
K_TRASH = 80000
E_PAD = 327680
CH = 128
NBUF = 8
L1_CHUNKS = 160
L2_CHUNKS = 80
NG1 = L1_CHUNKS // NBUF
NG2 = L2_CHUNKS // NBUF
NTILES = 16
SLAB = KS // NTILES
ZROWS = 313
BN = 400
GRID = N // BN

f32 = jnp.float32
i32 = jnp.int32


def _mesh():
    return plsc.VectorSubcoreMesh(core_axis_name="c", subcore_axis_name="s")


_SC_PARAMS = pltpu.CompilerParams(needs_layout_passes=False,
                                  use_tc_tiling_on_sc=False)


def _cnt_body(keyw, cnt32, keyv, cnt):
    c = lax.axis_index("c")
    s = lax.axis_index("s")
    w = s * 2 + c
    pltpu.sync_copy(keyw.at[w], keyv)
    z16 = jnp.zeros((16,), f32)

    def zb(i, carry):
        cnt[pl.ds(i * 16, 16)] = z16
        return carry

    lax.fori_loop(0, KS // 16, zb, 0)
    ones = jnp.ones((16,), f32)

    def body(i, carry):
        k = keyv[pl.ds(i * 16, 16)]
        plsc.addupdate_scatter(cnt, [k], ones)
        return carry

    lax.fori_loop(0, (E_PAD // 32) // 16, body, 0)
    pltpu.sync_copy(cnt, cnt32.at[w])


def _cnt_call(keyw):
    return pl.kernel(
        _cnt_body,
        out_type=jax.ShapeDtypeStruct((32, KS), f32),
        mesh=_mesh(),
        compiler_params=_SC_PARAMS,
        scratch_types=[
            pltpu.VMEM((E_PAD // 32,), i32),
            pltpu.VMEM((KS,), f32),
        ],
    )(keyw)


def _ring(tbl, k3, i3, acc, kidx, gidx, rows, gsem, ssem, ng, off):
    def load_idx(g1, par):
        pltpu.sync_copy(k3.at[pl.ds(g1 * NBUF, NBUF)], kidx.at[par])
        pltpu.sync_copy(i3.at[pl.ds(g1 * NBUF, NBUF)], gidx.at[par])
        if off is not None:
            for b in range(NBUF):
                for t in range(CH // 16):
                    sl = pl.ds(t * 16, 16)
                    gidx[par, b, sl] = gidx[par, b, sl] + off

    def issue_gathers(par):
        for b in range(NBUF):
            pltpu.async_copy(tbl.at[gidx.at[par, b]], rows.at[par, b],
                             gsem[par])

    def wait_gathers(par):
        for b in range(NBUF):
            pltpu.make_async_copy(tbl.at[gidx.at[par, b]], rows.at[par, b],
                                  gsem[par]).wait()

    def issue_scatters(par):
        for b in range(NBUF):
            pltpu.async_copy(rows.at[par, b], acc.at[kidx.at[par, b]],
                             ssem[par], add=True)

    def wait_scatters(par):
        for b in range(NBUF):
            pltpu.make_async_copy(rows.at[par, b], acc.at[kidx.at[par, b]],
                                  ssem[par]).wait()

    load_idx(0, 0)
    issue_gathers(0)

    def sup_body(sup, carry):
        for par in range(2):
            g = 2 * sup + par
            wait_gathers(par)

            @pl.when(g >= 1)
            def _():
                wait_scatters(1 - par)

            @pl.when(g <= ng - 2)
            def _():
                load_idx(g + 1, 1 - par)
                issue_gathers(1 - par)

            issue_scatters(par)
        return carry

    lax.fori_loop(0, ng // 2, sup_body, 0)
    wait_scatters((ng - 1) % 2)


def _zero_zbuf(zbuf):
    z16 = jnp.zeros((16,), f32)

    def zb(i, carry):
        zbuf[i, :] = z16
        return carry

    lax.fori_loop(0, ZROWS, zb, 0)


def _zero_slab(zbuf, acc, slab):
    for k in range(SLAB // ZROWS):
        pltpu.sync_copy(zbuf, acc.at[pl.ds(slab + k * ZROWS, ZROWS)])


def _l1_body(tbl, key3, idx3, u8, kidx, gidx, rows, zbuf, acc,
             gs0, gs1, ss0, ss1):
    c = lax.axis_index("c")
    s = lax.axis_index("s")
    k3 = key3.at[s]
    i3 = idx3.at[s]
    gsem = (gs0, gs1)
    ssem = (ss0, ss1)

    _zero_zbuf(zbuf)
    slab = s * SLAB
    _zero_slab(zbuf, acc, slab)
    plsc.subcore_barrier()

    for p in range(4):
        off = c * jnp.int32(4 * TROW) + jnp.int32(p * TROW)
        _ring(tbl, k3, i3, acc, kidx, gidx, rows, gsem, ssem, NG1, off)
        plsc.subcore_barrier()
        pltpu.sync_copy(acc.at[pl.ds(slab, SLAB)],
                        u8.at[4 * c + p, pl.ds(slab, SLAB)])
        if p < 3:
            _zero_slab(zbuf, acc, slab)
        plsc.subcore_barrier()


def _l1_call(tbl1, key3, idx3):
    return pl.kernel(
        _l1_body,
        out_type=jax.ShapeDtypeStruct((8, KS, 16), f32),
        mesh=_mesh(),
        compiler_params=_SC_PARAMS,
        scratch_types=[
            pltpu.VMEM((2, NBUF, CH), i32),
            pltpu.VMEM((2, NBUF, CH), i32),
            pltpu.VMEM((2, NBUF, CH, 16), f32),
            pltpu.VMEM((ZROWS, 16), f32),
            pltpu.VMEM_SHARED((KS, 16), f32),
            pltpu.SemaphoreType.DMA,
            pltpu.SemaphoreType.DMA,
            pltpu.SemaphoreType.DMA,
            pltpu.SemaphoreType.DMA,
        ],
    )(tbl1, key3, idx3)


def _l2_body(tbl, key3, idx3, part, kidx, gidx, rows, zbuf, acc,
             gs0, gs1, ss0, ss1):
    c = lax.axis_index("c")
    s = lax.axis_index("s")
    w = s * 2 + c
    k3 = key3.at[w]
    i3 = idx3.at[w]
    gsem = (gs0, gs1)
    ssem = (ss0, ss1)

    _zero_zbuf(zbuf)
    slab = s * SLAB
    _zero_slab(zbuf, acc, slab)
    plsc.subcore_barrier()

    _ring(tbl, k3, i3, acc, kidx, gidx, rows, gsem, ssem, NG2, None)
    plsc.subcore_barrier()
    pltpu.sync_copy(acc.at[pl.ds(slab, SLAB)],
                    part.at[c, pl.ds(slab, SLAB)])


def _l2_call(tbl2, key3, idx3):
    return pl.kernel(
        _l2_body,
        out_type=jax.ShapeDtypeStruct((2, KS, 16), f32),
        mesh=_mesh(),
        compiler_params=_SC_PARAMS,
        scratch_types=[
            pltpu.VMEM((2, NBUF, CH), i32),
            pltpu.VMEM((2, NBUF, CH), i32),
            pltpu.VMEM((2, NBUF, CH, 16), f32),
            pltpu.VMEM((ZROWS, 16), f32),
            pltpu.VMEM_SHARED((KS, 16), f32),
            pltpu.SemaphoreType.DMA,
            pltpu.SemaphoreType.DMA,
            pltpu.SemaphoreType.DMA,
            pltpu.SemaphoreType.DMA,
        ],
    )(tbl2, key3, idx3)


def _tc1_body(u8_ref, inv_ref, root1_ref, wcat_ref, xw_ref, dense_ref):
    invrep = inv_ref[...]
    cols = []
    for j in range(8):
        wj = u8_ref[j] * invrep
        hj = wj[:, 0:16]
        for r in range(1, 8):
            hj = hj + wj[:, r * 16:(r + 1) * 16]
        cols.append(hj)
    h = jnp.concatenate(cols, axis=1)
    h = jnp.maximum(h + root1_ref[...], 0.0)
    prod = jnp.dot(h, wcat_ref[...], preferred_element_type=f32)
    xw_ref[...] = prod[:, :H]
    dense_ref[...] = prod[:, H:]


def _tc1_call(u8r, invrep, root1b, wcat):
    return pl.pallas_call(
        _tc1_body,
        grid=(GRID,),
        in_specs=[
            pl.BlockSpec((8, BN, H), lambda i: (0, i, 0)),
            pl.BlockSpec((BN, H), lambda i: (i, 0)),
            pl.BlockSpec((BN, H), lambda i: (i, 0)),
            pl.BlockSpec((H, H + L), lambda i: (0, 0)),
        ],
        out_specs=[
            pl.BlockSpec((BN, H), lambda i: (i, 0)),
            pl.BlockSpec((BN, L), lambda i: (i, 0)),
        ],
        out_shape=[
            jax.ShapeDtypeStruct((N, H), f32),
            jax.ShapeDtypeStruct((N, L), f32),
        ],
    )(u8r, invrep, root1b, wcat)


def _tc2_body(part_ref, inv_ref, dense_ref, out_ref):
    invrep = inv_ref[...]
    pp = part_ref[...]
    v = (pp[0] + pp[1]) * invrep
    agg = jnp.zeros((BN, 16), f32)
    for r in range(8):
        agg = agg + v[:, r * 16:(r + 1) * 16]
    out_ref[...] = jax.nn.sigmoid(agg + dense_ref[...])


def _tc2_call(partr, invrep, dense):
    return pl.pallas_call(
        _tc2_body,
        grid=(GRID,),
        in_specs=[
            pl.BlockSpec((2, BN, H), lambda i: (0, i, 0)),
            pl.BlockSpec((BN, H), lambda i: (i, 0)),
            pl.BlockSpec((BN, L), lambda i: (i, 0)),
        ],
        out_specs=pl.BlockSpec((BN, L), lambda i: (i, 0)),
        out_shape=jax.ShapeDtypeStruct((N, L), f32),
    )(partr, invrep, dense)


def kernel(edge_index, edge_type, weight1, root1, bias1, weight2, root2,
           bias2):
    src = edge_index[0]
    dst = edge_index[1]
    rel = edge_type
    key = dst * R + rel
    idx1 = rel * N + src
    idx2 = src * R + rel
    pad = E_PAD - E
    key_p = jnp.concatenate([key, jnp.full((pad,), K_TRASH, i32)])
    idx1_p = jnp.concatenate([idx1, jnp.zeros((pad,), i32)])
    idx2_p = jnp.concatenate([idx2, jnp.zeros((pad,), i32)])

    tbl1 = weight1.reshape(TROW, 8, 16).transpose(1, 0, 2).reshape(8 * TROW, 16)

    cnt32 = _cnt_call(key_p.reshape(32, E_PAD // 32))
    u8 = _l1_call(tbl1,
                  key_p.reshape(NTILES, L1_CHUNKS, CH),
                  idx1_p.reshape(NTILES, L1_CHUNKS, CH))

    u8r = u8.reshape(8, KS * 16 // H, H)
    cnt = cnt32.sum(axis=0)[:N * R]
    inv = 1.0 / jnp.maximum(cnt, 1.0)
    invrep = jnp.broadcast_to(
        inv.reshape(N, R)[:, :, None], (N, R, 16)).reshape(N, H)
    root1b = root1 + bias1[None, :]
    wcat = jnp.concatenate(
        [weight2.transpose(1, 0, 2).reshape(H, R * L), root2], axis=1)
    xw, dense = _tc1_call(u8r, invrep, root1b, wcat)

    part = _l2_call(xw.reshape(N * R, L),
                    key_p.reshape(32, L2_CHUNKS, CH),
                    idx2_p.reshape(32, L2_CHUNKS, CH))
    partr = part.reshape(2, KS * 16 // H, H)
    out = _tc2_call(partr, invrep, dense + bias2[None, :])
    return out


# speedup vs baseline: 25.4736x; 1.8721x over previous
import jax
import jax.numpy as jnp
from jax import lax
from jax.experimental import pallas as pl
from jax.experimental.pallas import tpu as pltpu
from jax.experimental.pallas import tpu_sc as plsc

N = 10000
R = 8
H = 128
L = 16
E = 320000

TROW = R * N
KS = 80128
K_TRASH = 80000
E_PAD = 327680
CH = 128
NBUF = 8
L1_CHUNKS = 160
L2_CHUNKS = 80
NG1 = L1_CHUNKS // NBUF
NG2 = L2_CHUNKS // NBUF
NTILES = 16
SLAB = KS // NTILES
ZROWS = 313
BN = 400
GRID = N // BN

f32 = jnp.float32
i32 = jnp.int32


def _mesh():
    return plsc.VectorSubcoreMesh(core_axis_name="c", subcore_axis_name="s")


_SC_PARAMS = pltpu.CompilerParams(needs_layout_passes=False,
                                  use_tc_tiling_on_sc=False)


def _cnt_body(keyw, cnt32, keyv, cnt):
    c = lax.axis_index("c")
    s = lax.axis_index("s")
    w = s * 2 + c
    pltpu.sync_copy(keyw.at[w], keyv)
    z16 = jnp.zeros((16,), f32)

    def zb(i, carry):
        cnt[pl.ds(i * 16, 16)] = z16
        return carry

    lax.fori_loop(0, KS // 16, zb, 0)
    ones = jnp.ones((16,), f32)

    def body(i, carry):
        k = keyv[pl.ds(i * 16, 16)]
        plsc.addupdate_scatter(cnt, [k], ones)
        return carry

    lax.fori_loop(0, (E_PAD // 32) // 16, body, 0)
    pltpu.sync_copy(cnt, cnt32.at[w])


def _cnt_call(keyw):
    return pl.kernel(
        _cnt_body,
        out_type=jax.ShapeDtypeStruct((32, KS), f32),
        mesh=_mesh(),
        compiler_params=_SC_PARAMS,
        scratch_types=[
            pltpu.VMEM((E_PAD // 32,), i32),
            pltpu.VMEM((KS,), f32),
        ],
    )(keyw)


def _ring(tbl, k3, i3, acc, kidx, gidx, rows, gsem, ssem, ng, off):
    def load_idx(g1, par):
        pltpu.sync_copy(k3.at[pl.ds(g1 * NBUF, NBUF)], kidx.at[par])
        pltpu.sync_copy(i3.at[pl.ds(g1 * NBUF, NBUF)], gidx.at[par])
        if off is not None:
            for b in range(NBUF):
                for t in range(CH // 16):
                    sl = pl.ds(t * 16, 16)
                    gidx[par, b, sl] = gidx[par, b, sl] + off

    def issue_gathers(par):
        for b in range(NBUF):
            pltpu.async_copy(tbl.at[gidx.at[par, b]], rows.at[par, b],
                             gsem[par])

    def wait_gathers(par):
        for b in range(NBUF):
            pltpu.make_async_copy(tbl.at[gidx.at[par, b]], rows.at[par, b],
                                  gsem[par]).wait()

    def issue_scatters(par):
        for b in range(NBUF):
            pltpu.async_copy(rows.at[par, b], acc.at[kidx.at[par, b]],
                             ssem[par], add=True)

    def wait_scatters(par):
        for b in range(NBUF):
            pltpu.make_async_copy(rows.at[par, b], acc.at[kidx.at[par, b]],
                                  ssem[par]).wait()

    load_idx(0, 0)
    issue_gathers(0)

    def sup_body(sup, carry):
        for par in range(2):
            g = 2 * sup + par
            wait_gathers(par)

            @pl.when(g >= 1)
            def _():
                wait_scatters(1 - par)

            @pl.when(g <= ng - 2)
            def _():
                load_idx(g + 1, 1 - par)
                issue_gathers(1 - par)

            issue_scatters(par)
        return carry

    lax.fori_loop(0, ng // 2, sup_body, 0)
    wait_scatters((ng - 1) % 2)


def _zero_zbuf(zbuf):
    z16 = jnp.zeros((16,), f32)

    def zb(i, carry):
        zbuf[i, :] = z16
        return carry

    lax.fori_loop(0, ZROWS, zb, 0)


def _zero_slab(zbuf, acc, slab):
    for k in range(SLAB // ZROWS):
        pltpu.sync_copy(zbuf, acc.at[pl.ds(slab + k * ZROWS, ZROWS)])


def _l1_body(tbl, key3, idx3, u8, kidx, gidx, rows, zbuf, acc,
             gs0, gs1, ss0, ss1):
    c = lax.axis_index("c")
    s = lax.axis_index("s")
    k3 = key3.at[s]
    i3 = idx3.at[s]
    gsem = (gs0, gs1)
    ssem = (ss0, ss1)

    _zero_zbuf(zbuf)
    slab = s * SLAB
    _zero_slab(zbuf, acc, slab)
    plsc.subcore_barrier()

    for p in range(4):
        off = c * jnp.int32(4) + jnp.int32(p)
        _ring(tbl, k3, i3, acc, kidx, gidx, rows, gsem, ssem, NG1, off)
        plsc.subcore_barrier()
        pltpu.sync_copy(acc.at[pl.ds(slab, SLAB)],
                        u8.at[4 * c + p, pl.ds(slab, SLAB)])
        if p < 3:
            _zero_slab(zbuf, acc, slab)
        plsc.subcore_barrier()


def _l1_call(tbl1, key3, idx3):
    return pl.kernel(
        _l1_body,
        out_type=jax.ShapeDtypeStruct((8, KS, 16), f32),
        mesh=_mesh(),
        compiler_params=_SC_PARAMS,
        scratch_types=[
            pltpu.VMEM((2, NBUF, CH), i32),
            pltpu.VMEM((2, NBUF, CH), i32),
            pltpu.VMEM((2, NBUF, CH, 16), f32),
            pltpu.VMEM((ZROWS, 16), f32),
            pltpu.VMEM_SHARED((KS, 16), f32),
            pltpu.SemaphoreType.DMA,
            pltpu.SemaphoreType.DMA,
            pltpu.SemaphoreType.DMA,
            pltpu.SemaphoreType.DMA,
        ],
    )(tbl1, key3, idx3)


def _l2_body(tbl, key3, idx3, part, kidx, gidx, rows, zbuf, acc,
             gs0, gs1, ss0, ss1):
    c = lax.axis_index("c")
    s = lax.axis_index("s")
    w = s * 2 + c
    k3 = key3.at[w]
    i3 = idx3.at[w]
    gsem = (gs0, gs1)
    ssem = (ss0, ss1)

    _zero_zbuf(zbuf)
    slab = s * SLAB
    _zero_slab(zbuf, acc, slab)
    plsc.subcore_barrier()

    _ring(tbl, k3, i3, acc, kidx, gidx, rows, gsem, ssem, NG2, None)
    plsc.subcore_barrier()
    pltpu.sync_copy(acc.at[pl.ds(slab, SLAB)],
                    part.at[c, pl.ds(slab, SLAB)])


def _l2_call(tbl2, key3, idx3):
    return pl.kernel(
        _l2_body,
        out_type=jax.ShapeDtypeStruct((2, KS, 16), f32),
        mesh=_mesh(),
        compiler_params=_SC_PARAMS,
        scratch_types=[
            pltpu.VMEM((2, NBUF, CH), i32),
            pltpu.VMEM((2, NBUF, CH), i32),
            pltpu.VMEM((2, NBUF, CH, 16), f32),
            pltpu.VMEM((ZROWS, 16), f32),
            pltpu.VMEM_SHARED((KS, 16), f32),
            pltpu.SemaphoreType.DMA,
            pltpu.SemaphoreType.DMA,
            pltpu.SemaphoreType.DMA,
            pltpu.SemaphoreType.DMA,
        ],
    )(tbl2, key3, idx3)


def _tc1_body(u8_ref, inv_ref, root1_ref, wcat_ref, xw_ref, dense_ref):
    invrep = inv_ref[...]
    cols = []
    for j in range(8):
        wj = u8_ref[j] * invrep
        hj = wj[:, 0:16]
        for r in range(1, 8):
            hj = hj + wj[:, r * 16:(r + 1) * 16]
        cols.append(hj)
    h = jnp.concatenate(cols, axis=1)
    h = jnp.maximum(h + root1_ref[...], 0.0)
    prod = jnp.dot(h, wcat_ref[...], preferred_element_type=f32)
    xw_ref[...] = prod[:, :H]
    dense_ref[...] = prod[:, H:]


def _tc1_call(u8r, invrep, root1b, wcat):
    return pl.pallas_call(
        _tc1_body,
        grid=(GRID,),
        in_specs=[
            pl.BlockSpec((8, BN, H), lambda i: (0, i, 0)),
            pl.BlockSpec((BN, H), lambda i: (i, 0)),
            pl.BlockSpec((BN, H), lambda i: (i, 0)),
            pl.BlockSpec((H, H + L), lambda i: (0, 0)),
        ],
        out_specs=[
            pl.BlockSpec((BN, H), lambda i: (i, 0)),
            pl.BlockSpec((BN, L), lambda i: (i, 0)),
        ],
        out_shape=[
            jax.ShapeDtypeStruct((N, H), f32),
            jax.ShapeDtypeStruct((N, L), f32),
        ],
    )(u8r, invrep, root1b, wcat)


def _tc2_body(part_ref, inv_ref, dense_ref, out_ref):
    invrep = inv_ref[...]
    pp = part_ref[...]
    v = (pp[0] + pp[1]) * invrep
    agg = jnp.zeros((BN, 16), f32)
    for r in range(8):
        agg = agg + v[:, r * 16:(r + 1) * 16]
    out_ref[...] = jax.nn.sigmoid(agg + dense_ref[...])


def _tc2_call(partr, invrep, dense):
    return pl.pallas_call(
        _tc2_body,
        grid=(GRID,),
        in_specs=[
            pl.BlockSpec((2, BN, H), lambda i: (0, i, 0)),
            pl.BlockSpec((BN, H), lambda i: (i, 0)),
            pl.BlockSpec((BN, L), lambda i: (i, 0)),
        ],
        out_specs=pl.BlockSpec((BN, L), lambda i: (i, 0)),
        out_shape=jax.ShapeDtypeStruct((N, L), f32),
    )(partr, invrep, dense)


def kernel(edge_index, edge_type, weight1, root1, bias1, weight2, root2,
           bias2):
    src = edge_index[0]
    dst = edge_index[1]
    rel = edge_type
    key = dst * R + rel
    idx1 = (rel * N + src) * 8
    idx2 = src * R + rel
    pad = E_PAD - E
    ar = jnp.arange(pad, dtype=i32)
    key_p = jnp.concatenate([key, K_TRASH + (ar % (KS - K_TRASH))])
    idx1_p = jnp.concatenate([idx1, (ar % TROW) * 8])
    idx2_p = jnp.concatenate([idx2, ar % (N * R)])

    tbl1 = weight1.reshape(8 * TROW, 16)

    cnt32 = _cnt_call(key_p.reshape(32, E_PAD // 32))
    u8 = _l1_call(tbl1,
                  key_p.reshape(NTILES, L1_CHUNKS, CH),
                  idx1_p.reshape(NTILES, L1_CHUNKS, CH))

    u8r = u8.reshape(8, KS * 16 // H, H)
    cnt = cnt32.sum(axis=0)[:N * R]
    inv = 1.0 / jnp.maximum(cnt, 1.0)
    invrep = jnp.broadcast_to(
        inv.reshape(N, R)[:, :, None], (N, R, 16)).reshape(N, H)
    root1b = root1 + bias1[None, :]
    wcat = jnp.concatenate(
        [weight2.transpose(1, 0, 2).reshape(H, R * L), root2], axis=1)
    xw, dense = _tc1_call(u8r, invrep, root1b, wcat)

    part = _l2_call(xw.reshape(N * R, L),
                    key_p.reshape(32, L2_CHUNKS, CH),
                    idx2_p.reshape(32, L2_CHUNKS, CH))
    partr = part.reshape(2, KS * 16 // H, H)
    out = _tc2_call(partr, invrep, dense + bias2[None, :])
    return out


# speedup vs baseline: 28.0821x; 1.1024x over previous
import numpy as np

import jax
import jax.numpy as jnp
from jax import lax
from jax.experimental import pallas as pl
from jax.experimental.pallas import tpu as pltpu
from jax.experimental.pallas import tpu_sc as plsc

N = 10000
R = 8
H = 128
L = 16
E = 320000

TROW = R * N
KS = 80128
K_TRASH = 80000
E_PAD = 327680
CH = 128
NBUF = 8
L1_CHUNKS = 160
L2_CHUNKS = 80
NG1 = L1_CHUNKS // NBUF
NG2 = L2_CHUNKS // NBUF
NTILES = 16
SLAB = KS // NTILES
ZROWS = 313
BN = 400
GRID = N // BN

f32 = jnp.float32
i32 = jnp.int32


def _mesh():
    return plsc.VectorSubcoreMesh(core_axis_name="c", subcore_axis_name="s")


_SC_PARAMS = pltpu.CompilerParams(needs_layout_passes=False,
                                  use_tc_tiling_on_sc=False)


def _cnt_body(keyw, cnt32, keyv, cnt):
    c = lax.axis_index("c")
    s = lax.axis_index("s")
    w = s * 2 + c
    pltpu.sync_copy(keyw.at[w], keyv)
    z16 = jnp.zeros((16,), f32)

    def zb(i, carry):
        cnt[pl.ds(i * 16, 16)] = z16
        return carry

    lax.fori_loop(0, KS // 16, zb, 0)
    ones = jnp.ones((16,), f32)

    def body(i, carry):
        k = keyv[pl.ds(i * 16, 16)]
        plsc.addupdate_scatter(cnt, [k], ones)
        return carry

    lax.fori_loop(0, (E_PAD // 32) // 16, body, 0)
    pltpu.sync_copy(cnt, cnt32.at[w])


def _cnt_call(keyw):
    return pl.kernel(
        _cnt_body,
        out_type=jax.ShapeDtypeStruct((32, KS), f32),
        mesh=_mesh(),
        compiler_params=_SC_PARAMS,
        scratch_types=[
            pltpu.VMEM((E_PAD // 32,), i32),
            pltpu.VMEM((KS,), f32),
        ],
    )(keyw)


def _ring(tbl, k3, i3, acc, kidx, gidx, rows, gsem, ssem, ng, off):
    def load_idx(g1, par):
        pltpu.sync_copy(k3.at[pl.ds(g1 * NBUF, NBUF)], kidx.at[par])
        pltpu.sync_copy(i3.at[pl.ds(g1 * NBUF, NBUF)], gidx.at[par])
        if off is not None:
            for b in range(NBUF):
                for t in range(CH // 16):
                    sl = pl.ds(t * 16, 16)
                    gidx[par, b, sl] = gidx[par, b, sl] + off

    def issue_gathers(par):
        for b in range(NBUF):
            pltpu.async_copy(tbl.at[gidx.at[par, b]], rows.at[par, b],
                             gsem[par])

    def wait_gathers(par):
        for b in range(NBUF):
            pltpu.make_async_copy(tbl.at[gidx.at[par, b]], rows.at[par, b],
                                  gsem[par]).wait()

    def issue_scatters(par):
        for b in range(NBUF):
            pltpu.async_copy(rows.at[par, b], acc.at[kidx.at[par, b]],
                             ssem[par], add=True)

    def wait_scatters(par):
        for b in range(NBUF):
            pltpu.make_async_copy(rows.at[par, b], acc.at[kidx.at[par, b]],
                                  ssem[par]).wait()

    load_idx(0, 0)
    issue_gathers(0)

    def sup_body(sup, carry):
        for par in range(2):
            g = 2 * sup + par
            wait_gathers(par)

            @pl.when(g >= 1)
            def _():
                wait_scatters(1 - par)

            @pl.when(g <= ng - 2)
            def _():
                load_idx(g + 1, 1 - par)
                issue_gathers(1 - par)

            issue_scatters(par)
        return carry

    lax.fori_loop(0, ng // 2, sup_body, 0)
    wait_scatters((ng - 1) % 2)


def _zero_zbuf(zbuf):
    z16 = jnp.zeros((16,), f32)

    def zb(i, carry):
        zbuf[i, :] = z16
        return carry

    lax.fori_loop(0, ZROWS, zb, 0)


def _zero_slab(zbuf, acc, slab):
    for k in range(SLAB // ZROWS):
        pltpu.sync_copy(zbuf, acc.at[pl.ds(slab + k * ZROWS, ZROWS)])


def _l1_body(tbl, key3, idx3, u8, kidx, gidx, rows, zbuf, acc,
             gs0, gs1, ss0, ss1):
    c = lax.axis_index("c")
    s = lax.axis_index("s")
    k3 = key3.at[s]
    i3 = idx3.at[s]
    gsem = (gs0, gs1)
    ssem = (ss0, ss1)

    _zero_zbuf(zbuf)
    slab = s * SLAB
    _zero_slab(zbuf, acc, slab)
    plsc.subcore_barrier()

    for p in range(4):
        off = c * jnp.int32(4) + jnp.int32(p)
        _ring(tbl, k3, i3, acc, kidx, gidx, rows, gsem, ssem, NG1, off)
        plsc.subcore_barrier()
        pltpu.sync_copy(acc.at[pl.ds(slab, SLAB)],
                        u8.at[4 * c + p, pl.ds(slab, SLAB)])
        if p < 3:
            _zero_slab(zbuf, acc, slab)
        plsc.subcore_barrier()


def _l1_call(tbl1, key3, idx3):
    return pl.kernel(
        _l1_body,
        out_type=jax.ShapeDtypeStruct((8, KS, 16), f32),
        mesh=_mesh(),
        compiler_params=_SC_PARAMS,
        scratch_types=[
            pltpu.VMEM((2, NBUF, CH), i32),
            pltpu.VMEM((2, NBUF, CH), i32),
            pltpu.VMEM((2, NBUF, CH, 16), f32),
            pltpu.VMEM((ZROWS, 16), f32),
            pltpu.VMEM_SHARED((KS, 16), f32),
            pltpu.SemaphoreType.DMA,
            pltpu.SemaphoreType.DMA,
            pltpu.SemaphoreType.DMA,
            pltpu.SemaphoreType.DMA,
        ],
    )(tbl1, key3, idx3)


def _l2_body(tbl, key3, idx3, part, kidx, gidx, rows, zbuf, acc,
             gs0, gs1, ss0, ss1):
    c = lax.axis_index("c")
    s = lax.axis_index("s")
    w = s * 2 + c
    k3 = key3.at[w]
    i3 = idx3.at[w]
    gsem = (gs0, gs1)
    ssem = (ss0, ss1)

    _zero_zbuf(zbuf)
    slab = s * SLAB
    _zero_slab(zbuf, acc, slab)
    plsc.subcore_barrier()

    _ring(tbl, k3, i3, acc, kidx, gidx, rows, gsem, ssem, NG2, None)
    plsc.subcore_barrier()
    pltpu.sync_copy(acc.at[pl.ds(slab, SLAB)],
                    part.at[c, pl.ds(slab, SLAB)])


def _l2_call(tbl2, key3, idx3):
    return pl.kernel(
        _l2_body,
        out_type=jax.ShapeDtypeStruct((2, KS, 16), f32),
        mesh=_mesh(),
        compiler_params=_SC_PARAMS,
        scratch_types=[
            pltpu.VMEM((2, NBUF, CH), i32),
            pltpu.VMEM((2, NBUF, CH), i32),
            pltpu.VMEM((2, NBUF, CH, 16), f32),
            pltpu.VMEM((ZROWS, 16), f32),
            pltpu.VMEM_SHARED((KS, 16), f32),
            pltpu.SemaphoreType.DMA,
            pltpu.SemaphoreType.DMA,
            pltpu.SemaphoreType.DMA,
            pltpu.SemaphoreType.DMA,
        ],
    )(tbl2, key3, idx3)


_S1_np = np.zeros((8, H, H), np.float32)
for _j in range(8):
    for _r in range(8):
        for _o in range(16):
            _S1_np[_j, _r * 16 + _o, _j * 16 + _o] = 1.0
_S2_np = np.zeros((H, L), np.float32)
for _r in range(8):
    for _o in range(16):
        _S2_np[_r * 16 + _o, _o] = 1.0


def _tc1_body(u8_ref, inv_ref, root1_ref, wcat_ref, s1_ref, xw_ref,
              dense_ref):
    invrep = inv_ref[...]
    h = root1_ref[...]
    for j in range(8):
        h = h + jnp.dot(u8_ref[j] * invrep, s1_ref[j],
                        preferred_element_type=f32,
                        precision=lax.Precision.HIGHEST)
    h = jnp.maximum(h, 0.0)
    prod = jnp.dot(h, wcat_ref[...], preferred_element_type=f32)
    xw_ref[...] = prod[:, :H]
    dense_ref[...] = prod[:, H:]


def _tc1_call(u8r, invrep, root1b, wcat):
    return pl.pallas_call(
        _tc1_body,
        grid=(GRID,),
        in_specs=[
            pl.BlockSpec((8, BN, H), lambda i: (0, i, 0)),
            pl.BlockSpec((BN, H), lambda i: (i, 0)),
            pl.BlockSpec((BN, H), lambda i: (i, 0)),
            pl.BlockSpec((H, H + L), lambda i: (0, 0)),
            pl.BlockSpec((8, H, H), lambda i: (0, 0, 0)),
        ],
        out_specs=[
            pl.BlockSpec((BN, H), lambda i: (i, 0)),
            pl.BlockSpec((BN, L), lambda i: (i, 0)),
        ],
        out_shape=[
            jax.ShapeDtypeStruct((N, H), f32),
            jax.ShapeDtypeStruct((N, L), f32),
        ],
    )(u8r, invrep, root1b, wcat, jnp.asarray(_S1_np))


def _tc2_body(part_ref, inv_ref, dense_ref, s2_ref, out_ref):
    invrep = inv_ref[...]
    pp = part_ref[...]
    v = (pp[0] + pp[1]) * invrep
    agg = jnp.dot(v, s2_ref[...], preferred_element_type=f32,
                  precision=lax.Precision.HIGHEST)
    out_ref[...] = jax.nn.sigmoid(agg + dense_ref[...])


def _tc2_call(partr, invrep, dense):
    return pl.pallas_call(
        _tc2_body,
        grid=(GRID,),
        in_specs=[
            pl.BlockSpec((2, BN, H), lambda i: (0, i, 0)),
            pl.BlockSpec((BN, H), lambda i: (i, 0)),
            pl.BlockSpec((BN, L), lambda i: (i, 0)),
            pl.BlockSpec((H, L), lambda i: (0, 0)),
        ],
        out_specs=pl.BlockSpec((BN, L), lambda i: (i, 0)),
        out_shape=jax.ShapeDtypeStruct((N, L), f32),
    )(partr, invrep, dense, jnp.asarray(_S2_np))


def kernel(edge_index, edge_type, weight1, root1, bias1, weight2, root2,
           bias2):
    src = edge_index[0]
    dst = edge_index[1]
    rel = edge_type
    key = dst * R + rel
    idx1 = (rel * N + src) * 8
    idx2 = src * R + rel
    pad = E_PAD - E
    ar = jnp.arange(pad, dtype=i32)
    key_p = jnp.concatenate([key, K_TRASH + (ar % (KS - K_TRASH))])
    idx1_p = jnp.concatenate([idx1, (ar % TROW) * 8])
    idx2_p = jnp.concatenate([idx2, ar % (N * R)])

    tbl1 = weight1.reshape(8 * TROW, 16)

    cnt32 = _cnt_call(key_p.reshape(32, E_PAD // 32))
    u8 = _l1_call(tbl1,
                  key_p.reshape(NTILES, L1_CHUNKS, CH),
                  idx1_p.reshape(NTILES, L1_CHUNKS, CH))

    u8r = u8.reshape(8, KS * 16 // H, H)
    cnt = cnt32.sum(axis=0)[:N * R]
    inv = 1.0 / jnp.maximum(cnt, 1.0)
    invrep = jnp.broadcast_to(
        inv.reshape(N, R)[:, :, None], (N, R, 16)).reshape(N, H)
    root1b = root1 + bias1[None, :]
    wcat = jnp.concatenate(
        [weight2.transpose(1, 0, 2).reshape(H, R * L), root2], axis=1)
    xw, dense = _tc1_call(u8r, invrep, root1b, wcat)

    part = _l2_call(xw.reshape(N * R, L),
                    key_p.reshape(32, L2_CHUNKS, CH),
                    idx2_p.reshape(32, L2_CHUNKS, CH))
    partr = part.reshape(2, KS * 16 // H, H)
    out = _tc2_call(partr, invrep, dense + bias2[None, :])
    return out


# speedup vs baseline: 28.1500x; 1.0024x over previous
import numpy as np

import jax
import jax.numpy as jnp
from jax import lax
from jax.experimental import pallas as pl
from jax.experimental.pallas import tpu as pltpu
from jax.experimental.pallas import tpu_sc as plsc

N = 10000
R = 8
H = 128
L = 16
E = 320000

TROW = R * N
KS = 80128
K_TRASH = 80000
E_PAD = 327680
CH = 128
NBUF = 8
L1_CHUNKS = 160
L2_CHUNKS = 80
NG1 = L1_CHUNKS // NBUF
NG2 = L2_CHUNKS // NBUF
NTILES = 16
SLAB = KS // NTILES
ZROWS = 313
BN = 400
GRID = N // BN

f32 = jnp.float32
i32 = jnp.int32


def _mesh():
    return plsc.VectorSubcoreMesh(core_axis_name="c", subcore_axis_name="s")


_SC_PARAMS = pltpu.CompilerParams(needs_layout_passes=False,
                                  use_tc_tiling_on_sc=False)


EW = E // 32


def _cnt_body(dstw, relw, cnt32, dstv, relv, cnt):
    c = lax.axis_index("c")
    s = lax.axis_index("s")
    w = s * 2 + c
    pltpu.sync_copy(dstw.at[w], dstv)
    pltpu.sync_copy(relw.at[w], relv)
    z16 = jnp.zeros((16,), f32)

    def zb(i, carry):
        cnt[pl.ds(i * 16, 16)] = z16
        return carry

    lax.fori_loop(0, KS // 16, zb, 0)
    ones = jnp.ones((16,), f32)
    eight = jnp.full((16,), 8, i32)

    def body(i, carry):
        k = dstv[pl.ds(i * 16, 16)] * eight + relv[pl.ds(i * 16, 16)]
        plsc.addupdate_scatter(cnt, [k], ones)
        return carry

    lax.fori_loop(0, EW // 16, body, 0)
    pltpu.sync_copy(cnt, cnt32.at[w])


def _cnt_call(dstw, relw):
    return pl.kernel(
        _cnt_body,
        out_type=jax.ShapeDtypeStruct((32, KS), f32),
        mesh=_mesh(),
        compiler_params=_SC_PARAMS,
        scratch_types=[
            pltpu.VMEM((EW,), i32),
            pltpu.VMEM((EW,), i32),
            pltpu.VMEM((KS,), f32),
        ],
    )(dstw, relw)


def _ring(tbl, k3, i3, acc, kidx, gidx, rows, gsem, ssem, ng, off):
    def load_idx(g1, par):
        pltpu.sync_copy(k3.at[pl.ds(g1 * NBUF, NBUF)], kidx.at[par])
        pltpu.sync_copy(i3.at[pl.ds(g1 * NBUF, NBUF)], gidx.at[par])
        if off is not None:
            for b in range(NBUF):
                for t in range(CH // 16):
                    sl = pl.ds(t * 16, 16)
                    gidx[par, b, sl] = gidx[par, b, sl] + off

    def issue_gathers(par):
        for b in range(NBUF):
            pltpu.async_copy(tbl.at[gidx.at[par, b]], rows.at[par, b],
                             gsem[par])

    def wait_gathers(par):
        for b in range(NBUF):
            pltpu.make_async_copy(tbl.at[gidx.at[par, b]], rows.at[par, b],
                                  gsem[par]).wait()

    def issue_scatters(par):
        for b in range(NBUF):
            pltpu.async_copy(rows.at[par, b], acc.at[kidx.at[par, b]],
                             ssem[par], add=True)

    def wait_scatters(par):
        for b in range(NBUF):
            pltpu.make_async_copy(rows.at[par, b], acc.at[kidx.at[par, b]],
                                  ssem[par]).wait()

    load_idx(0, 0)
    issue_gathers(0)

    def sup_body(sup, carry):
        for par in range(2):
            g = 2 * sup + par
            wait_gathers(par)

            @pl.when(g >= 1)
            def _():
                wait_scatters(1 - par)

            @pl.when(g <= ng - 2)
            def _():
                load_idx(g + 1, 1 - par)
                issue_gathers(1 - par)

            issue_scatters(par)
        return carry

    lax.fori_loop(0, ng // 2, sup_body, 0)
    wait_scatters((ng - 1) % 2)


def _zero_zbuf(zbuf):
    z16 = jnp.zeros((16,), f32)

    def zb(i, carry):
        zbuf[i, :] = z16
        return carry

    lax.fori_loop(0, ZROWS, zb, 0)


def _zero_slab(zbuf, acc, slab):
    for k in range(SLAB // ZROWS):
        pltpu.sync_copy(zbuf, acc.at[pl.ds(slab + k * ZROWS, ZROWS)])


def _l1_body(tbl, key3, idx3, u8, kidx, gidx, rows, zbuf, acc,
             gs0, gs1, ss0, ss1):
    c = lax.axis_index("c")
    s = lax.axis_index("s")
    k3 = key3.at[s]
    i3 = idx3.at[s]
    gsem = (gs0, gs1)
    ssem = (ss0, ss1)

    _zero_zbuf(zbuf)
    slab = s * SLAB
    _zero_slab(zbuf, acc, slab)
    plsc.subcore_barrier()

    for p in range(4):
        off = c * jnp.int32(4) + jnp.int32(p)
        _ring(tbl, k3, i3, acc, kidx, gidx, rows, gsem, ssem, NG1, off)
        plsc.subcore_barrier()
        pltpu.sync_copy(acc.at[pl.ds(slab, SLAB)],
                        u8.at[4 * c + p, pl.ds(slab, SLAB)])
        if p < 3:
            _zero_slab(zbuf, acc, slab)
        plsc.subcore_barrier()


def _l1_call(tbl1, key3, idx3):
    return pl.kernel(
        _l1_body,
        out_type=jax.ShapeDtypeStruct((8, KS, 16), f32),
        mesh=_mesh(),
        compiler_params=_SC_PARAMS,
        scratch_types=[
            pltpu.VMEM((2, NBUF, CH), i32),
            pltpu.VMEM((2, NBUF, CH), i32),
            pltpu.VMEM((2, NBUF, CH, 16), f32),
            pltpu.VMEM((ZROWS, 16), f32),
            pltpu.VMEM_SHARED((KS, 16), f32),
            pltpu.SemaphoreType.DMA,
            pltpu.SemaphoreType.DMA,
            pltpu.SemaphoreType.DMA,
            pltpu.SemaphoreType.DMA,
        ],
    )(tbl1, key3, idx3)


def _l2_body(tbl, key3, idx3, part, kidx, gidx, rows, zbuf, acc,
             gs0, gs1, ss0, ss1):
    c = lax.axis_index("c")
    s = lax.axis_index("s")
    w = s * 2 + c
    k3 = key3.at[w]
    i3 = idx3.at[w]
    gsem = (gs0, gs1)
    ssem = (ss0, ss1)

    _zero_zbuf(zbuf)
    slab = s * SLAB
    _zero_slab(zbuf, acc, slab)
    plsc.subcore_barrier()

    _ring(tbl, k3, i3, acc, kidx, gidx, rows, gsem, ssem, NG2, None)
    plsc.subcore_barrier()
    pltpu.sync_copy(acc.at[pl.ds(slab, SLAB)],
                    part.at[c, pl.ds(slab, SLAB)])


def _l2_call(tbl2, key3, idx3):
    return pl.kernel(
        _l2_body,
        out_type=jax.ShapeDtypeStruct((2, KS, 16), f32),
        mesh=_mesh(),
        compiler_params=_SC_PARAMS,
        scratch_types=[
            pltpu.VMEM((2, NBUF, CH), i32),
            pltpu.VMEM((2, NBUF, CH), i32),
            pltpu.VMEM((2, NBUF, CH, 16), f32),
            pltpu.VMEM((ZROWS, 16), f32),
            pltpu.VMEM_SHARED((KS, 16), f32),
            pltpu.SemaphoreType.DMA,
            pltpu.SemaphoreType.DMA,
            pltpu.SemaphoreType.DMA,
            pltpu.SemaphoreType.DMA,
        ],
    )(tbl2, key3, idx3)


_S1_np = np.zeros((8, H, H), np.float32)
for _j in range(8):
    for _r in range(8):
        for _o in range(16):
            _S1_np[_j, _r * 16 + _o, _j * 16 + _o] = 1.0
_S2_np = np.zeros((H, L), np.float32)
for _r in range(8):
    for _o in range(16):
        _S2_np[_r * 16 + _o, _o] = 1.0


def _tc1_body(u8_ref, inv_ref, root1_ref, wcat_ref, s1_ref, xw_ref,
              dense_ref):
    invrep = inv_ref[...]
    h = root1_ref[...]
    for j in range(8):
        h = h + jnp.dot(u8_ref[j] * invrep, s1_ref[j],
                        preferred_element_type=f32,
                        precision=lax.Precision.HIGHEST)
    h = jnp.maximum(h, 0.0)
    prod = jnp.dot(h, wcat_ref[...], preferred_element_type=f32)
    xw_ref[...] = prod[:, :H]
    dense_ref[...] = prod[:, H:]


def _tc1_call(u8r, invrep, root1b, wcat):
    return pl.pallas_call(
        _tc1_body,
        grid=(GRID,),
        in_specs=[
            pl.BlockSpec((8, BN, H), lambda i: (0, i, 0)),
            pl.BlockSpec((BN, H), lambda i: (i, 0)),
            pl.BlockSpec((BN, H), lambda i: (i, 0)),
            pl.BlockSpec((H, H + L), lambda i: (0, 0)),
            pl.BlockSpec((8, H, H), lambda i: (0, 0, 0)),
        ],
        out_specs=[
            pl.BlockSpec((BN, H), lambda i: (i, 0)),
            pl.BlockSpec((BN, L), lambda i: (i, 0)),
        ],
        out_shape=[
            jax.ShapeDtypeStruct((N, H), f32),
            jax.ShapeDtypeStruct((N, L), f32),
        ],
    )(u8r, invrep, root1b, wcat, jnp.asarray(_S1_np))


def _tc2_body(part_ref, inv_ref, dense_ref, s2_ref, out_ref):
    invrep = inv_ref[...]
    pp = part_ref[...]
    v = (pp[0] + pp[1]) * invrep
    agg = jnp.dot(v, s2_ref[...], preferred_element_type=f32,
                  precision=lax.Precision.HIGHEST)
    out_ref[...] = jax.nn.sigmoid(agg + dense_ref[...])


def _tc2_call(partr, invrep, dense):
    return pl.pallas_call(
        _tc2_body,
        grid=(GRID,),
        in_specs=[
            pl.BlockSpec((2, BN, H), lambda i: (0, i, 0)),
            pl.BlockSpec((BN, H), lambda i: (i, 0)),
            pl.BlockSpec((BN, L), lambda i: (i, 0)),
            pl.BlockSpec((H, L), lambda i: (0, 0)),
        ],
        out_specs=pl.BlockSpec((BN, L), lambda i: (i, 0)),
        out_shape=jax.ShapeDtypeStruct((N, L), f32),
    )(partr, invrep, dense, jnp.asarray(_S2_np))


def kernel(edge_index, edge_type, weight1, root1, bias1, weight2, root2,
           bias2):
    src = edge_index[0]
    dst = edge_index[1]
    rel = edge_type
    key = dst * R + rel
    idx1 = (rel * N + src) * 8
    idx2 = src * R + rel
    pad = E_PAD - E
    ar = jnp.arange(pad, dtype=i32)
    key_p = jnp.concatenate([key, K_TRASH + (ar % (KS - K_TRASH))])
    idx1_p = jnp.concatenate([idx1, (ar % TROW) * 8])
    idx2_p = jnp.concatenate([idx2, ar % (N * R)])

    tbl1 = weight1.reshape(8 * TROW, 16)

    cnt32 = _cnt_call(dst.reshape(32, EW), rel.reshape(32, EW))
    u8 = _l1_call(tbl1,
                  key_p.reshape(NTILES, L1_CHUNKS, CH),
                  idx1_p.reshape(NTILES, L1_CHUNKS, CH))

    u8r = u8.reshape(8, KS * 16 // H, H)
    cnt = cnt32.sum(axis=0)[:N * R]
    inv = 1.0 / jnp.maximum(cnt, 1.0)
    invrep = jnp.broadcast_to(
        inv.reshape(N, R)[:, :, None], (N, R, 16)).reshape(N, H)
    root1b = root1 + bias1[None, :]
    wcat = jnp.concatenate(
        [weight2.transpose(1, 0, 2).reshape(H, R * L), root2], axis=1)
    xw, dense = _tc1_call(u8r, invrep, root1b, wcat)

    part = _l2_call(xw.reshape(N * R, L),
                    key_p.reshape(32, L2_CHUNKS, CH),
                    idx2_p.reshape(32, L2_CHUNKS, CH))
    partr = part.reshape(2, KS * 16 // H, H)
    out = _tc2_call(partr, invrep, dense + bias2[None, :])
    return out
